# baseline (speedup 1.0000x reference)
import functools

import jax
import jax.numpy as jnp
from jax import lax
from jax.experimental import pallas as pl
from jax.experimental.pallas import tpu as pltpu
from jax.experimental.pallas import tpu_sc as plsc

N = 10000
E = 320000
D = 128
HID = 128
BASIS = 128
WIDE = 4
NLAYERS = 4

NC = 2
NS = 16
NW = NC * NS
LANES = 16

EPW = E // NW

CBLK = 96
CNB = EPW // CBLK
CTAIL = EPW - CNB * CBLK

PBLK = 128
PNB = EPW // PBLK
PTAIL = EPW - PNB * PBLK

SUB_ROWS = 624
ZCHUNK = 48
DCHUNK = 104


def _gelu(x):
    return 0.5 * x * (1.0 + lax.erf(x * 0.7071067811865476))


def _mesh():
    return plsc.VectorSubcoreMesh(core_axis_name="c", subcore_axis_name="s")


def _sc_pos_diff(pos128, send, recv):
    @functools.partial(
        pl.kernel,
        out_type=jax.ShapeDtypeStruct((E, HID), jnp.float32),
        mesh=_mesh(),
        scratch_types=[
            pltpu.VMEM((EPW,), jnp.int32),
            pltpu.VMEM((EPW,), jnp.int32),
            pltpu.VMEM((PBLK, HID), jnp.float32),
            pltpu.VMEM((PBLK, HID), jnp.float32),
            pltpu.VMEM((PBLK, HID), jnp.float32),
            pltpu.VMEM((PBLK, HID), jnp.float32),
            pltpu.SemaphoreType.DMA,
            pltpu.SemaphoreType.DMA,
            pltpu.SemaphoreType.DMA,
            pltpu.SemaphoreType.DMA,
        ],
    )
    def k(pos_hbm, send_hbm, recv_hbm, out_hbm, sidx_all, ridx_all,
          sa0, sa1, ra0, ra1, g0, g1, w0, w1):
        c = lax.axis_index("c")
        s = lax.axis_index("s")
        wid = s * NC + c
        e0 = wid * EPW

        sbufs = (sa0, sa1)
        rbufs = (ra0, ra1)
        gsems = (g0, g1)
        wsems = (w0, w1)

        pltpu.sync_copy(send_hbm.at[pl.ds(e0, EPW)], sidx_all)
        pltpu.sync_copy(recv_hbm.at[pl.ds(e0, EPW)], ridx_all)

        def issue(m, q):
            off = m * PBLK
            pltpu.async_copy(pos_hbm.at[sidx_all.at[pl.ds(off, PBLK)]],
                             sbufs[q], gsems[q])
            pltpu.async_copy(pos_hbm.at[ridx_all.at[pl.ds(off, PBLK)]],
                             rbufs[q], gsems[q])

        def wait_g(q):
            pltpu.make_async_copy(pos_hbm.at[pl.ds(0, PBLK)], sbufs[q],
                                  gsems[q]).wait()
            pltpu.make_async_copy(pos_hbm.at[pl.ds(0, PBLK)], rbufs[q],
                                  gsems[q]).wait()

        def wait_w(q):
            pltpu.make_async_copy(pos_hbm.at[pl.ds(0, PBLK)], sbufs[q],
                                  wsems[q]).wait()

        def sub(q, nrow):
            sb, rb = sbufs[q], rbufs[q]

            @plsc.parallel_loop(0, nrow, unroll=4)
            def _(r):
                sl = (r, pl.ds(0, LANES))
                sb[sl] = sb[sl] - rb[sl]

        def step(m, p):
            q = 1 - p

            @pl.when(m >= 1)
            def _():
                wait_w(q)

            @pl.when(m + 1 < PNB)
            def _():
                issue(m + 1, q)

            wait_g(p)
            sub(p, PBLK)
            pltpu.async_copy(sbufs[p], out_hbm.at[pl.ds(e0 + m * PBLK, PBLK)],
                             wsems[p])

        issue(0, 0)

        @pl.loop(0, PNB // 2)
        def _(j):
            step(2 * j, 0)
            step(2 * j + 1, 1)

        toff = PNB * PBLK
        pltpu.sync_copy(pos_hbm.at[sidx_all.at[pl.ds(toff, PTAIL)]],
                        sa0.at[pl.ds(0, PTAIL)])
        pltpu.sync_copy(pos_hbm.at[ridx_all.at[pl.ds(toff, PTAIL)]],
                        ra0.at[pl.ds(0, PTAIL)])
        sub(0, PTAIL)
        pltpu.sync_copy(sa0.at[pl.ds(0, PTAIL)],
                        out_hbm.at[pl.ds(e0 + toff, PTAIL)])
        wait_w(1)

    return k(pos128, send, recv)


def _sc_conv(h, kern, send, recv):
    @functools.partial(
        pl.kernel,
        out_type=jax.ShapeDtypeStruct((2 * N, HID), jnp.float32),
        mesh=_mesh(),
        scratch_types=[
            pltpu.VMEM((CBLK,), jnp.int32),
            pltpu.VMEM((CBLK,), jnp.int32),
            pltpu.VMEM((CBLK,), jnp.int32),
            pltpu.VMEM((CBLK,), jnp.int32),
            pltpu.VMEM((CBLK,), jnp.int32),
            pltpu.VMEM((CBLK,), jnp.int32),
            pltpu.VMEM((CTAIL,), jnp.int32),
            pltpu.VMEM((CTAIL,), jnp.int32),
            pltpu.VMEM((CBLK, HID), jnp.float32),
            pltpu.VMEM((CBLK, HID), jnp.float32),
            pltpu.VMEM((CBLK, HID), jnp.float32),
            pltpu.VMEM((CBLK, HID), jnp.float32),
            pltpu.SemaphoreType.DMA,
            pltpu.SemaphoreType.DMA,
            pltpu.SemaphoreType.DMA,
            pltpu.SemaphoreType.DMA,
            pltpu.SemaphoreType.DMA,
            pltpu.SemaphoreType.DMA,
            pltpu.VMEM_SHARED((N, HID), jnp.float32),
        ],
    )
    def k(h_hbm, k_hbm, send_hbm, recv_hbm, out_hbm,
          si0, si1, ri0, ri1, rs0, rs1, sit, rit, h0, h1, k0, k1,
          i0, i1, g0, g1, s0, s1, acc):
        c = lax.axis_index("c")
        s = lax.axis_index("s")
        wid = s * NC + c
        e0 = wid * EPW

        sidx = (si0, si1)
        ridx = (ri0, ri1)
        rsnap = (rs0, rs1)
        hbufs = (h0, h1)
        kbufs = (k0, k1)
        isems = (i0, i1)
        gsems = (g0, g1)
        ssems = (s0, s1)

        @pl.loop(0, ZCHUNK)
        def _(r):
            for j in range(HID // LANES):
                h0[r, pl.ds(j * LANES, LANES)] = jnp.zeros((LANES,), jnp.float32)

        @pl.loop(0, SUB_ROWS // ZCHUNK)
        def _(j):
            pltpu.sync_copy(h0.at[pl.ds(0, ZCHUNK)],
                            acc.at[pl.ds(s * SUB_ROWS + j * ZCHUNK, ZCHUNK)])

        @pl.when(s == NS - 1)
        def _():
            pltpu.sync_copy(h0.at[pl.ds(0, 16)],
                            acc.at[pl.ds(NS * SUB_ROWS, 16)])

        plsc.subcore_barrier()

        def issue_idx(m, q):
            off = e0 + m * CBLK
            pltpu.async_copy(send_hbm.at[pl.ds(off, CBLK)], sidx[q], isems[q])
            pltpu.async_copy(recv_hbm.at[pl.ds(off, CBLK)], ridx[q], isems[q])

        def wait_idx(q):
            pltpu.make_async_copy(send_hbm.at[pl.ds(0, CBLK)], sidx[q],
                                  isems[q]).wait()
            pltpu.make_async_copy(recv_hbm.at[pl.ds(0, CBLK)], ridx[q],
                                  isems[q]).wait()

        def issue_g(m, q):
            pltpu.async_copy(h_hbm.at[sidx[q]], hbufs[q], gsems[q])
            pltpu.async_copy(k_hbm.at[pl.ds(e0 + m * CBLK, CBLK)], kbufs[q],
                             gsems[q])

        def wait_g(q):
            pltpu.make_async_copy(h_hbm.at[pl.ds(0, CBLK)], hbufs[q],
                                  gsems[q]).wait()
            pltpu.make_async_copy(k_hbm.at[pl.ds(0, CBLK)], kbufs[q],
                                  gsems[q]).wait()

        def wait_s(q):
            pltpu.make_async_copy(k_hbm.at[pl.ds(0, CBLK)], kbufs[q],
                                  ssems[q]).wait()

        def mul(q, nrow):
            kb, hb = kbufs[q], hbufs[q]

            @plsc.parallel_loop(0, nrow, unroll=4)
            def _(r):
                for j in range(HID // LANES):
                    sl = (r, pl.ds(j * LANES, LANES))
                    kb[sl] = kb[sl] * hb[sl]

        def step(m, p):
            q = 1 - p

            @pl.when(m >= 1)
            def _():
                wait_s(q)

            @pl.when(m + 1 < CNB)
            def _():
                wait_idx(q)
                issue_g(m + 1, q)

            wait_g(p)
            mul(p, CBLK)
            rsrc, rdst = ridx[p], rsnap[p]

            @plsc.parallel_loop(0, CBLK, step=LANES)
            def _(r):
                rdst[pl.ds(r, LANES)] = rsrc[pl.ds(r, LANES)]

            pltpu.async_copy(kbufs[p], acc.at[rsnap[p]], ssems[p], add=True)

            @pl.when(m + 2 < CNB)
            def _():
                issue_idx(m + 2, p)

        issue_idx(0, 0)
        issue_idx(1, 1)
        wait_idx(0)
        issue_g(0, 0)

        @pl.loop(0, CNB // 2)
        def _(j):
            step(2 * j, 0)
            step(2 * j + 1, 1)

        toff = e0 + CNB * CBLK
        pltpu.sync_copy(send_hbm.at[pl.ds(toff, CTAIL)], sit)
        pltpu.sync_copy(recv_hbm.at[pl.ds(toff, CTAIL)], rit)
        pltpu.sync_copy(h_hbm.at[sit], h0.at[pl.ds(0, CTAIL)])
        pltpu.sync_copy(k_hbm.at[pl.ds(toff, CTAIL)], k0.at[pl.ds(0, CTAIL)])
        mul(0, CTAIL)
        pltpu.sync_copy(k0.at[pl.ds(0, CTAIL)], acc.at[rit], add=True)
        wait_s(1)

        plsc.subcore_barrier()

        @pl.loop(0, SUB_ROWS // DCHUNK)
        def _(j):
            row0 = s * SUB_ROWS + j * DCHUNK
            pltpu.sync_copy(acc.at[pl.ds(row0, DCHUNK)],
                            out_hbm.at[pl.ds(c * N + row0, DCHUNK)])

        @pl.when(s == NS - 1)
        def _():
            pltpu.sync_copy(acc.at[pl.ds(NS * SUB_ROWS, 16)],
                            out_hbm.at[pl.ds(c * N + NS * SUB_ROWS, 16)])

    return k(h, kern, send, recv)


def _tc_basis_u(diff, Wb1, bb1, Wb2, bb2):
    EB = 3200
    grid = (E // EB,)

    def body(diff_ref, wb1_ref, bb1_ref, wb2_ref, bb2_ref, u_ref):
        df = diff_ref[...]
        d2 = jnp.sum(df * df, axis=1, keepdims=True) + 1e-12
        d = jnp.sqrt(d2)
        dd = d * d
        ddd = dd * d
        t = (d * wb1_ref[0:1, :] + dd * wb1_ref[1:2, :] + ddd * wb1_ref[2:3, :]
             + bb1_ref[...])
        t = _gelu(t)
        u_ref[...] = _gelu(
            jnp.dot(t, wb2_ref[...], preferred_element_type=jnp.float32)
            + bb2_ref[...])

    return pl.pallas_call(
        body,
        grid=grid,
        in_specs=[
            pl.BlockSpec((EB, HID), lambda i: (i, 0)),
            pl.BlockSpec((8, BASIS), lambda i: (0, 0)),
            pl.BlockSpec((1, BASIS), lambda i: (0, 0)),
            pl.BlockSpec((BASIS, BASIS), lambda i: (0, 0)),
            pl.BlockSpec((1, BASIS), lambda i: (0, 0)),
        ],
        out_specs=pl.BlockSpec((EB, BASIS), lambda i: (i, 0)),
        out_shape=jax.ShapeDtypeStruct((E, BASIS), jnp.float32),
    )(diff, Wb1, bb1, Wb2, bb2)


def _tc_kmat(u, Wki):
    EB = 6400
    grid = (E // EB,)

    def body(u_ref, wk_ref, k_ref):
        k_ref[...] = jnp.dot(u_ref[...], wk_ref[...],
                             preferred_element_type=jnp.float32)

    return pl.pallas_call(
        body,
        grid=grid,
        in_specs=[
            pl.BlockSpec((EB, BASIS), lambda i: (i, 0)),
            pl.BlockSpec((BASIS, HID), lambda i: (0, 0)),
        ],
        out_specs=pl.BlockSpec((EB, HID), lambda i: (i, 0)),
        out_shape=jax.ShapeDtypeStruct((E, HID), jnp.float32),
    )(u, Wki)


def _tc_embed(x, W_embed):
    def body(x_ref, w_ref, o_ref):
        o_ref[...] = jnp.dot(x_ref[...], w_ref[...],
                             preferred_element_type=jnp.float32)

    return pl.pallas_call(
        body, out_shape=jax.ShapeDtypeStruct((N, HID), jnp.float32)
    )(x, W_embed)


def _tc_node_update(aggs, h, gamma, beta, W1, b1, W2, b2, Wro=None, bro=None):
    readout = Wro is not None

    def body(aggs_ref, h_ref, g_ref, be_ref, w1_ref, b1_ref, w2_ref, b2_ref,
             *rest):
        out_ref = rest[-1]
        agg = aggs_ref[0:N, :] + aggs_ref[N:2 * N, :]
        mu = jnp.mean(agg, axis=-1, keepdims=True)
        xc = agg - mu
        var = jnp.mean(xc * xc, axis=-1, keepdims=True)
        z = xc / jnp.sqrt(var + 1e-5) * g_ref[...] + be_ref[...]
        z = _gelu(jnp.dot(z, w1_ref[...], preferred_element_type=jnp.float32)
                  + b1_ref[...])
        z = jnp.dot(z, w2_ref[...], preferred_element_type=jnp.float32) + b2_ref[...]
        hn = h_ref[...] + z
        if readout:
            wro_ref, bro_ref = rest[0], rest[1]
            out_ref[...] = (jnp.dot(hn, wro_ref[...],
                                    preferred_element_type=jnp.float32)
                            + bro_ref[...])
        else:
            out_ref[...] = hn

    args = [aggs, h, gamma, beta, W1, b1, W2, b2]
    odim = D if readout else HID
    if readout:
        args += [Wro, bro]
    return pl.pallas_call(
        body, out_shape=jax.ShapeDtypeStruct((N, odim), jnp.float32)
    )(*args)


def kernel(x, pos, edge_index, batch, W_embed, Wb1, bb1, Wb2, bb2, Wk,
           gamma, beta, W1, b1, W2, b2, Wro, bro):
    send = edge_index[0].astype(jnp.int32)
    recv = edge_index[1].astype(jnp.int32)
    pos128 = jnp.zeros((N, HID), jnp.float32).at[:, :3].set(pos)
    Wb1p = jnp.zeros((8, BASIS), jnp.float32).at[:3].set(Wb1)

    diff = _sc_pos_diff(pos128, send, recv)
    u = _tc_basis_u(diff, Wb1p, bb1[None, :], Wb2, bb2[None, :])
    kerns = [_tc_kmat(u, Wk[i]) for i in range(NLAYERS)]
    h = _tc_embed(x, W_embed)
    for i in range(NLAYERS):
        aggs = _sc_conv(h, kerns[i], send, recv)
        last = i == NLAYERS - 1
        h = _tc_node_update(
            aggs, h, gamma[i][None, :], beta[i][None, :],
            W1[i], b1[i][None, :], W2[i], b2[i][None, :],
            Wro if last else None, bro[None, :] if last else None)
    return h


# speedup vs baseline: 1.1646x; 1.1646x over previous
import functools

import jax
import jax.numpy as jnp
from jax import lax
from jax.experimental import pallas as pl
from jax.experimental.pallas import tpu as pltpu
from jax.experimental.pallas import tpu_sc as plsc

N = 10000
E = 320000
D = 128
HID = 128
BASIS = 128
WIDE = 4
NLAYERS = 4

NC = 2
NS = 16
NW = NC * NS
LANES = 16

EPW = E // NW

CBLK = 96
CNB = EPW // CBLK
CTAIL = EPW - CNB * CBLK

PBLK = 128
PNB = EPW // PBLK
PTAIL = EPW - PNB * PBLK

SUB_ROWS = 624
ZCHUNK = 48
DCHUNK = 104


def _gelu(x):
    return 0.5 * x * (1.0 + lax.erf(x * 0.7071067811865476))


def _mesh():
    return plsc.VectorSubcoreMesh(core_axis_name="c", subcore_axis_name="s")


def _sc_pos_diff(pos128, send, recv):
    @functools.partial(
        pl.kernel,
        out_type=jax.ShapeDtypeStruct((E, HID), jnp.float32),
        mesh=_mesh(),
        scratch_types=[
            pltpu.VMEM((EPW,), jnp.int32),
            pltpu.VMEM((EPW,), jnp.int32),
            pltpu.VMEM((PBLK, HID), jnp.float32),
            pltpu.VMEM((PBLK, HID), jnp.float32),
            pltpu.VMEM((PBLK, HID), jnp.float32),
            pltpu.VMEM((PBLK, HID), jnp.float32),
            pltpu.SemaphoreType.DMA,
            pltpu.SemaphoreType.DMA,
            pltpu.SemaphoreType.DMA,
            pltpu.SemaphoreType.DMA,
        ],
    )
    def k(pos_hbm, send_hbm, recv_hbm, out_hbm, sidx_all, ridx_all,
          sa0, sa1, ra0, ra1, g0, g1, w0, w1):
        c = lax.axis_index("c")
        s = lax.axis_index("s")
        wid = s * NC + c
        e0 = wid * EPW

        sbufs = (sa0, sa1)
        rbufs = (ra0, ra1)
        gsems = (g0, g1)
        wsems = (w0, w1)

        pltpu.sync_copy(send_hbm.at[pl.ds(e0, EPW)], sidx_all)
        pltpu.sync_copy(recv_hbm.at[pl.ds(e0, EPW)], ridx_all)

        def issue(m, q):
            off = m * PBLK
            pltpu.async_copy(pos_hbm.at[sidx_all.at[pl.ds(off, PBLK)]],
                             sbufs[q], gsems[q])
            pltpu.async_copy(pos_hbm.at[ridx_all.at[pl.ds(off, PBLK)]],
                             rbufs[q], gsems[q])

        def wait_g(q):
            pltpu.make_async_copy(pos_hbm.at[pl.ds(0, PBLK)], sbufs[q],
                                  gsems[q]).wait()
            pltpu.make_async_copy(pos_hbm.at[pl.ds(0, PBLK)], rbufs[q],
                                  gsems[q]).wait()

        def wait_w(q):
            pltpu.make_async_copy(pos_hbm.at[pl.ds(0, PBLK)], sbufs[q],
                                  wsems[q]).wait()

        def sub(q, nrow):
            sb, rb = sbufs[q], rbufs[q]

            @plsc.parallel_loop(0, nrow, unroll=4)
            def _(r):
                sl = (r, pl.ds(0, LANES))
                sb[sl] = sb[sl] - rb[sl]

        def step(m, p):
            q = 1 - p

            @pl.when(m >= 1)
            def _():
                wait_w(q)

            @pl.when(m + 1 < PNB)
            def _():
                issue(m + 1, q)

            wait_g(p)
            sub(p, PBLK)
            pltpu.async_copy(sbufs[p], out_hbm.at[pl.ds(e0 + m * PBLK, PBLK)],
                             wsems[p])

        issue(0, 0)

        @pl.loop(0, PNB // 2)
        def _(j):
            step(2 * j, 0)
            step(2 * j + 1, 1)

        toff = PNB * PBLK
        pltpu.sync_copy(pos_hbm.at[sidx_all.at[pl.ds(toff, PTAIL)]],
                        sa0.at[pl.ds(0, PTAIL)])
        pltpu.sync_copy(pos_hbm.at[ridx_all.at[pl.ds(toff, PTAIL)]],
                        ra0.at[pl.ds(0, PTAIL)])
        sub(0, PTAIL)
        pltpu.sync_copy(sa0.at[pl.ds(0, PTAIL)],
                        out_hbm.at[pl.ds(e0 + toff, PTAIL)])
        wait_w(1)

    return k(pos128, send, recv)


def _sc_conv(h, kern, send, recv):
    @functools.partial(
        pl.kernel,
        out_type=jax.ShapeDtypeStruct((2 * N, HID), jnp.float32),
        mesh=_mesh(),
        scratch_types=[
            pltpu.VMEM((CBLK,), jnp.int32),
            pltpu.VMEM((CBLK,), jnp.int32),
            pltpu.VMEM((CBLK,), jnp.int32),
            pltpu.VMEM((CBLK,), jnp.int32),
            pltpu.VMEM((CBLK,), jnp.int32),
            pltpu.VMEM((CBLK,), jnp.int32),
            pltpu.VMEM((CTAIL,), jnp.int32),
            pltpu.VMEM((CTAIL,), jnp.int32),
            pltpu.VMEM((CBLK, HID), jnp.float32),
            pltpu.VMEM((CBLK, HID), jnp.float32),
            pltpu.VMEM((CBLK, HID), jnp.float32),
            pltpu.VMEM((CBLK, HID), jnp.float32),
            pltpu.SemaphoreType.DMA,
            pltpu.SemaphoreType.DMA,
            pltpu.SemaphoreType.DMA,
            pltpu.SemaphoreType.DMA,
            pltpu.SemaphoreType.DMA,
            pltpu.SemaphoreType.DMA,
            pltpu.VMEM_SHARED((N, HID), jnp.float32),
        ],
    )
    def k(h_hbm, k_hbm, send_hbm, recv_hbm, out_hbm,
          si0, si1, ri0, ri1, rs0, rs1, sit, rit, h0, h1, k0, k1,
          i0, i1, g0, g1, s0, s1, acc):
        c = lax.axis_index("c")
        s = lax.axis_index("s")
        wid = s * NC + c
        e0 = wid * EPW

        sidx = (si0, si1)
        ridx = (ri0, ri1)
        rsnap = (rs0, rs1)
        hbufs = (h0, h1)
        kbufs = (k0, k1)
        isems = (i0, i1)
        gsems = (g0, g1)
        ssems = (s0, s1)

        @pl.loop(0, ZCHUNK)
        def _(r):
            for j in range(HID // LANES):
                h0[r, pl.ds(j * LANES, LANES)] = jnp.zeros((LANES,), jnp.float32)

        @pl.loop(0, SUB_ROWS // ZCHUNK)
        def _(j):
            pltpu.sync_copy(h0.at[pl.ds(0, ZCHUNK)],
                            acc.at[pl.ds(s * SUB_ROWS + j * ZCHUNK, ZCHUNK)])

        @pl.when(s == NS - 1)
        def _():
            pltpu.sync_copy(h0.at[pl.ds(0, 16)],
                            acc.at[pl.ds(NS * SUB_ROWS, 16)])

        plsc.subcore_barrier()

        def issue_idx(m, q):
            off = e0 + m * CBLK
            pltpu.async_copy(send_hbm.at[pl.ds(off, CBLK)], sidx[q], isems[q])
            pltpu.async_copy(recv_hbm.at[pl.ds(off, CBLK)], ridx[q], isems[q])

        def wait_idx(q):
            pltpu.make_async_copy(send_hbm.at[pl.ds(0, CBLK)], sidx[q],
                                  isems[q]).wait()
            pltpu.make_async_copy(recv_hbm.at[pl.ds(0, CBLK)], ridx[q],
                                  isems[q]).wait()

        def issue_g(m, q):
            pltpu.async_copy(h_hbm.at[sidx[q]], hbufs[q], gsems[q])
            pltpu.async_copy(k_hbm.at[pl.ds(e0 + m * CBLK, CBLK)], kbufs[q],
                             gsems[q])

        def wait_g(q):
            pltpu.make_async_copy(h_hbm.at[pl.ds(0, CBLK)], hbufs[q],
                                  gsems[q]).wait()
            pltpu.make_async_copy(k_hbm.at[pl.ds(0, CBLK)], kbufs[q],
                                  gsems[q]).wait()

        def wait_s(q):
            pltpu.make_async_copy(k_hbm.at[pl.ds(0, CBLK)], kbufs[q],
                                  ssems[q]).wait()

        def mul(q, nrow):
            kb, hb = kbufs[q], hbufs[q]

            @plsc.parallel_loop(0, nrow, unroll=4)
            def _(r):
                for j in range(HID // LANES):
                    sl = (r, pl.ds(j * LANES, LANES))
                    kb[sl] = kb[sl] * hb[sl]

        def step(m, p):
            q = 1 - p

            @pl.when(m >= 1)
            def _():
                wait_s(q)

            @pl.when(m + 1 < CNB)
            def _():
                wait_idx(q)
                issue_g(m + 1, q)

            wait_g(p)
            mul(p, CBLK)
            rsrc, rdst = ridx[p], rsnap[p]

            @plsc.parallel_loop(0, CBLK, step=LANES)
            def _(r):
                rdst[pl.ds(r, LANES)] = rsrc[pl.ds(r, LANES)]

            pltpu.async_copy(kbufs[p], acc.at[rsnap[p]], ssems[p], add=True)

            @pl.when(m + 2 < CNB)
            def _():
                issue_idx(m + 2, p)

        issue_idx(0, 0)
        issue_idx(1, 1)
        wait_idx(0)
        issue_g(0, 0)

        @pl.loop(0, CNB // 2)
        def _(j):
            step(2 * j, 0)
            step(2 * j + 1, 1)

        toff = e0 + CNB * CBLK
        pltpu.sync_copy(send_hbm.at[pl.ds(toff, CTAIL)], sit)
        pltpu.sync_copy(recv_hbm.at[pl.ds(toff, CTAIL)], rit)
        pltpu.sync_copy(h_hbm.at[sit], h0.at[pl.ds(0, CTAIL)])
        pltpu.sync_copy(k_hbm.at[pl.ds(toff, CTAIL)], k0.at[pl.ds(0, CTAIL)])
        mul(0, CTAIL)
        pltpu.sync_copy(k0.at[pl.ds(0, CTAIL)], acc.at[rit], add=True)
        wait_s(1)

        plsc.subcore_barrier()

        @pl.loop(0, SUB_ROWS // DCHUNK)
        def _(j):
            row0 = s * SUB_ROWS + j * DCHUNK
            pltpu.sync_copy(acc.at[pl.ds(row0, DCHUNK)],
                            out_hbm.at[pl.ds(c * N + row0, DCHUNK)])

        @pl.when(s == NS - 1)
        def _():
            pltpu.sync_copy(acc.at[pl.ds(NS * SUB_ROWS, 16)],
                            out_hbm.at[pl.ds(c * N + NS * SUB_ROWS, 16)])

    return k(h, kern, send, recv)


def _tc_edge_basis(diff, Wb1, bb1, Wb2, bb2, Wk):
    EB = 3200
    grid = (E // EB,)

    def body(diff_ref, wb1_ref, bb1_ref, wb2_ref, bb2_ref, wk_ref,
             k0_ref, k1_ref, k2_ref, k3_ref):
        df = diff_ref[...]
        d2 = jnp.sum(df * df, axis=1, keepdims=True) + 1e-12
        d = jnp.sqrt(d2)
        dd = d * d
        ddd = dd * d
        t = (d * wb1_ref[0:1, :] + dd * wb1_ref[1:2, :] + ddd * wb1_ref[2:3, :]
             + bb1_ref[...])
        t = _gelu(t)
        u = _gelu(jnp.dot(t, wb2_ref[...], preferred_element_type=jnp.float32)
                  + bb2_ref[...])
        outs = (k0_ref, k1_ref, k2_ref, k3_ref)
        for i in range(NLAYERS):
            outs[i][...] = jnp.dot(u, wk_ref[i],
                                   preferred_element_type=jnp.float32)

    out_sd = jax.ShapeDtypeStruct((E, HID), jnp.float32)
    return pl.pallas_call(
        body,
        grid=grid,
        in_specs=[
            pl.BlockSpec((EB, HID), lambda i: (i, 0)),
            pl.BlockSpec((8, BASIS), lambda i: (0, 0)),
            pl.BlockSpec((1, BASIS), lambda i: (0, 0)),
            pl.BlockSpec((BASIS, BASIS), lambda i: (0, 0)),
            pl.BlockSpec((1, BASIS), lambda i: (0, 0)),
            pl.BlockSpec((NLAYERS, BASIS, HID), lambda i: (0, 0, 0)),
        ],
        out_specs=[pl.BlockSpec((EB, HID), lambda i: (i, 0))] * NLAYERS,
        out_shape=[out_sd] * NLAYERS,
    )(diff, Wb1, bb1, Wb2, bb2, Wk)


def _tc_embed(x, W_embed):
    def body(x_ref, w_ref, o_ref):
        o_ref[...] = jnp.dot(x_ref[...], w_ref[...],
                             preferred_element_type=jnp.float32)

    return pl.pallas_call(
        body, out_shape=jax.ShapeDtypeStruct((N, HID), jnp.float32)
    )(x, W_embed)


def _tc_node_update(aggs, h, gamma, beta, W1, b1, W2, b2, Wro=None, bro=None):
    readout = Wro is not None

    def body(aggs_ref, h_ref, g_ref, be_ref, w1_ref, b1_ref, w2_ref, b2_ref,
             *rest):
        out_ref = rest[-1]
        agg = aggs_ref[0:N, :] + aggs_ref[N:2 * N, :]
        mu = jnp.mean(agg, axis=-1, keepdims=True)
        xc = agg - mu
        var = jnp.mean(xc * xc, axis=-1, keepdims=True)
        z = xc / jnp.sqrt(var + 1e-5) * g_ref[...] + be_ref[...]
        z = _gelu(jnp.dot(z, w1_ref[...], preferred_element_type=jnp.float32)
                  + b1_ref[...])
        z = jnp.dot(z, w2_ref[...], preferred_element_type=jnp.float32) + b2_ref[...]
        hn = h_ref[...] + z
        if readout:
            wro_ref, bro_ref = rest[0], rest[1]
            out_ref[...] = (jnp.dot(hn, wro_ref[...],
                                    preferred_element_type=jnp.float32)
                            + bro_ref[...])
        else:
            out_ref[...] = hn

    args = [aggs, h, gamma, beta, W1, b1, W2, b2]
    odim = D if readout else HID
    if readout:
        args += [Wro, bro]
    return pl.pallas_call(
        body, out_shape=jax.ShapeDtypeStruct((N, odim), jnp.float32)
    )(*args)


def kernel(x, pos, edge_index, batch, W_embed, Wb1, bb1, Wb2, bb2, Wk,
           gamma, beta, W1, b1, W2, b2, Wro, bro):
    send = edge_index[0].astype(jnp.int32)
    recv = edge_index[1].astype(jnp.int32)
    pos128 = jnp.zeros((N, HID), jnp.float32).at[:, :3].set(pos)
    Wb1p = jnp.zeros((8, BASIS), jnp.float32).at[:3].set(Wb1)

    diff = _sc_pos_diff(pos128, send, recv)
    kerns = _tc_edge_basis(diff, Wb1p, bb1[None, :], Wb2, bb2[None, :], Wk)
    h = _tc_embed(x, W_embed)
    for i in range(NLAYERS):
        aggs = _sc_conv(h, kerns[i], send, recv)
        last = i == NLAYERS - 1
        h = _tc_node_update(
            aggs, h, gamma[i][None, :], beta[i][None, :],
            W1[i], b1[i][None, :], W2[i], b2[i][None, :],
            Wro if last else None, bro[None, :] if last else None)
    return h


# speedup vs baseline: 1.1657x; 1.0009x over previous
import functools

import jax
import jax.numpy as jnp
from jax import lax
from jax.experimental import pallas as pl
from jax.experimental.pallas import tpu as pltpu
from jax.experimental.pallas import tpu_sc as plsc

N = 10000
E = 320000
D = 128
HID = 128
BASIS = 128
WIDE = 4
NLAYERS = 4

NC = 2
NS = 16
NW = NC * NS
LANES = 16

EPW = E // NW

CBLK = 96
CNB = EPW // CBLK
CTAIL = EPW - CNB * CBLK

PBLK = 128
PNB = EPW // PBLK
PTAIL = EPW - PNB * PBLK

SUB_ROWS = 624
ZCHUNK = 48
DCHUNK = 104


def _gelu(x):
    return 0.5 * x * (1.0 + lax.erf(x * 0.7071067811865476))


def _mesh():
    return plsc.VectorSubcoreMesh(core_axis_name="c", subcore_axis_name="s")


def _sc_pos_diff(pos128, send, recv):
    @functools.partial(
        pl.kernel,
        out_type=jax.ShapeDtypeStruct((E, HID), jnp.float32),
        mesh=_mesh(),
        scratch_types=[
            pltpu.VMEM((EPW,), jnp.int32),
            pltpu.VMEM((EPW,), jnp.int32),
            pltpu.VMEM((PBLK, HID), jnp.float32),
            pltpu.VMEM((PBLK, HID), jnp.float32),
            pltpu.VMEM((PBLK, HID), jnp.float32),
            pltpu.VMEM((PBLK, HID), jnp.float32),
            pltpu.SemaphoreType.DMA,
            pltpu.SemaphoreType.DMA,
            pltpu.SemaphoreType.DMA,
            pltpu.SemaphoreType.DMA,
        ],
    )
    def k(pos_hbm, send_hbm, recv_hbm, out_hbm, sidx_all, ridx_all,
          sa0, sa1, ra0, ra1, g0, g1, w0, w1):
        c = lax.axis_index("c")
        s = lax.axis_index("s")
        wid = s * NC + c
        e0 = wid * EPW

        sbufs = (sa0, sa1)
        rbufs = (ra0, ra1)
        gsems = (g0, g1)
        wsems = (w0, w1)

        pltpu.sync_copy(send_hbm.at[pl.ds(e0, EPW)], sidx_all)
        pltpu.sync_copy(recv_hbm.at[pl.ds(e0, EPW)], ridx_all)

        def issue(m, q):
            off = m * PBLK
            pltpu.async_copy(pos_hbm.at[sidx_all.at[pl.ds(off, PBLK)]],
                             sbufs[q], gsems[q])
            pltpu.async_copy(pos_hbm.at[ridx_all.at[pl.ds(off, PBLK)]],
                             rbufs[q], gsems[q])

        def wait_g(q):
            pltpu.make_async_copy(pos_hbm.at[pl.ds(0, PBLK)], sbufs[q],
                                  gsems[q]).wait()
            pltpu.make_async_copy(pos_hbm.at[pl.ds(0, PBLK)], rbufs[q],
                                  gsems[q]).wait()

        def wait_w(q):
            pltpu.make_async_copy(pos_hbm.at[pl.ds(0, PBLK)], sbufs[q],
                                  wsems[q]).wait()

        def sub(q, nrow):
            sb, rb = sbufs[q], rbufs[q]

            @plsc.parallel_loop(0, nrow, unroll=4)
            def _(r):
                sl = (r, pl.ds(0, LANES))
                sb[sl] = sb[sl] - rb[sl]

        def step(m, p):
            q = 1 - p

            @pl.when(m >= 1)
            def _():
                wait_w(q)

            @pl.when(m + 1 < PNB)
            def _():
                issue(m + 1, q)

            wait_g(p)
            sub(p, PBLK)
            pltpu.async_copy(sbufs[p], out_hbm.at[pl.ds(e0 + m * PBLK, PBLK)],
                             wsems[p])

        issue(0, 0)

        @pl.loop(0, PNB // 2)
        def _(j):
            step(2 * j, 0)
            step(2 * j + 1, 1)

        toff = PNB * PBLK
        pltpu.sync_copy(pos_hbm.at[sidx_all.at[pl.ds(toff, PTAIL)]],
                        sa0.at[pl.ds(0, PTAIL)])
        pltpu.sync_copy(pos_hbm.at[ridx_all.at[pl.ds(toff, PTAIL)]],
                        ra0.at[pl.ds(0, PTAIL)])
        sub(0, PTAIL)
        pltpu.sync_copy(sa0.at[pl.ds(0, PTAIL)],
                        out_hbm.at[pl.ds(e0 + toff, PTAIL)])
        wait_w(1)

    return k(pos128, send, recv)


def _sc_conv(h, kern, send, recv):
    @functools.partial(
        pl.kernel,
        out_type=jax.ShapeDtypeStruct((2 * N, HID), jnp.float32),
        mesh=_mesh(),
        scratch_types=[
            pltpu.VMEM((CBLK,), jnp.int32),
            pltpu.VMEM((CBLK,), jnp.int32),
            pltpu.VMEM((CBLK,), jnp.int32),
            pltpu.VMEM((CBLK,), jnp.int32),
            pltpu.VMEM((CBLK,), jnp.int32),
            pltpu.VMEM((CBLK,), jnp.int32),
            pltpu.VMEM((CTAIL,), jnp.int32),
            pltpu.VMEM((CTAIL,), jnp.int32),
            pltpu.VMEM((CBLK, HID), jnp.float32),
            pltpu.VMEM((CBLK, HID), jnp.float32),
            pltpu.VMEM((CBLK, HID), jnp.float32),
            pltpu.VMEM((CBLK, HID), jnp.float32),
            pltpu.SemaphoreType.DMA,
            pltpu.SemaphoreType.DMA,
            pltpu.SemaphoreType.DMA,
            pltpu.SemaphoreType.DMA,
            pltpu.SemaphoreType.DMA,
            pltpu.SemaphoreType.DMA,
            pltpu.VMEM_SHARED((N, HID), jnp.float32),
        ],
    )
    def k(h_hbm, k_hbm, send_hbm, recv_hbm, out_hbm,
          si0, si1, ri0, ri1, rs0, rs1, sit, rit, h0, h1, k0, k1,
          i0, i1, g0, g1, s0, s1, acc):
        c = lax.axis_index("c")
        s = lax.axis_index("s")
        wid = s * NC + c
        e0 = wid * EPW

        sidx = (si0, si1)
        ridx = (ri0, ri1)
        rsnap = (rs0, rs1)
        hbufs = (h0, h1)
        kbufs = (k0, k1)
        isems = (i0, i1)
        gsems = (g0, g1)
        ssems = (s0, s1)

        @pl.loop(0, ZCHUNK)
        def _(r):
            for j in range(HID // LANES):
                h0[r, pl.ds(j * LANES, LANES)] = jnp.zeros((LANES,), jnp.float32)

        @pl.loop(0, SUB_ROWS // ZCHUNK)
        def _(j):
            pltpu.sync_copy(h0.at[pl.ds(0, ZCHUNK)],
                            acc.at[pl.ds(s * SUB_ROWS + j * ZCHUNK, ZCHUNK)])

        @pl.when(s == NS - 1)
        def _():
            pltpu.sync_copy(h0.at[pl.ds(0, 16)],
                            acc.at[pl.ds(NS * SUB_ROWS, 16)])

        plsc.subcore_barrier()

        def issue_idx(m, q):
            off = e0 + m * CBLK
            pltpu.async_copy(send_hbm.at[pl.ds(off, CBLK)], sidx[q], isems[q])
            pltpu.async_copy(recv_hbm.at[pl.ds(off, CBLK)], ridx[q], isems[q])

        def wait_idx(q):
            pltpu.make_async_copy(send_hbm.at[pl.ds(0, CBLK)], sidx[q],
                                  isems[q]).wait()
            pltpu.make_async_copy(recv_hbm.at[pl.ds(0, CBLK)], ridx[q],
                                  isems[q]).wait()

        def issue_g(m, q):
            pltpu.async_copy(h_hbm.at[sidx[q]], hbufs[q], gsems[q])
            pltpu.async_copy(k_hbm.at[pl.ds(e0 + m * CBLK, CBLK)], kbufs[q],
                             gsems[q])

        def wait_g(q):
            pltpu.make_async_copy(h_hbm.at[pl.ds(0, CBLK)], hbufs[q],
                                  gsems[q]).wait()
            pltpu.make_async_copy(k_hbm.at[pl.ds(0, CBLK)], kbufs[q],
                                  gsems[q]).wait()

        def wait_s(q):
            pltpu.make_async_copy(k_hbm.at[pl.ds(0, CBLK)], kbufs[q],
                                  ssems[q]).wait()

        def mul(q, nrow):
            kb, hb = kbufs[q], hbufs[q]

            @plsc.parallel_loop(0, nrow, unroll=4)
            def _(r):
                for j in range(HID // LANES):
                    sl = (r, pl.ds(j * LANES, LANES))
                    kb[sl] = kb[sl] * hb[sl]

        def step(m, p):
            q = 1 - p

            @pl.when(m >= 1)
            def _():
                wait_s(q)

            @pl.when(m + 1 < CNB)
            def _():
                wait_idx(q)
                issue_g(m + 1, q)

            wait_g(p)
            mul(p, CBLK)
            rsrc, rdst = ridx[p], rsnap[p]

            @plsc.parallel_loop(0, CBLK, step=LANES)
            def _(r):
                rdst[pl.ds(r, LANES)] = rsrc[pl.ds(r, LANES)]

            pltpu.async_copy(kbufs[p], acc.at[rsnap[p]], ssems[p], add=True)

            @pl.when(m + 2 < CNB)
            def _():
                issue_idx(m + 2, p)

        issue_idx(0, 0)
        issue_idx(1, 1)
        wait_idx(0)
        issue_g(0, 0)

        @pl.loop(0, CNB // 2)
        def _(j):
            step(2 * j, 0)
            step(2 * j + 1, 1)

        toff = e0 + CNB * CBLK
        pltpu.sync_copy(send_hbm.at[pl.ds(toff, CTAIL)], sit)
        pltpu.sync_copy(recv_hbm.at[pl.ds(toff, CTAIL)], rit)
        pltpu.sync_copy(h_hbm.at[sit], h0.at[pl.ds(0, CTAIL)])
        pltpu.sync_copy(k_hbm.at[pl.ds(toff, CTAIL)], k0.at[pl.ds(0, CTAIL)])
        mul(0, CTAIL)
        pltpu.sync_copy(k0.at[pl.ds(0, CTAIL)], acc.at[rit], add=True)
        wait_s(1)

        plsc.subcore_barrier()

        @pl.loop(0, SUB_ROWS // DCHUNK)
        def _(j):
            row0 = s * SUB_ROWS + j * DCHUNK
            pltpu.sync_copy(acc.at[pl.ds(row0, DCHUNK)],
                            out_hbm.at[pl.ds(c * N + row0, DCHUNK)])

        @pl.when(s == NS - 1)
        def _():
            pltpu.sync_copy(acc.at[pl.ds(NS * SUB_ROWS, 16)],
                            out_hbm.at[pl.ds(c * N + NS * SUB_ROWS, 16)])

    return k(h, kern, send, recv)


def _tc_edge_basis(diff, Wb1, bb1, Wb2, bb2, Wk):
    EB = 3200
    grid = (E // EB,)

    def body(diff_ref, wb1_ref, bb1_ref, wb2_ref, bb2_ref, wk_ref,
             k0_ref, k1_ref, k2_ref, k3_ref):
        df = diff_ref[...]
        d2 = jnp.sum(df * df, axis=1, keepdims=True) + 1e-12
        d = jnp.sqrt(d2)
        dd = d * d
        ddd = dd * d
        t = (d * wb1_ref[0:1, :] + dd * wb1_ref[1:2, :] + ddd * wb1_ref[2:3, :]
             + bb1_ref[...])
        t = _gelu(t)
        u = _gelu(jnp.dot(t, wb2_ref[...], preferred_element_type=jnp.float32)
                  + bb2_ref[...])
        outs = (k0_ref, k1_ref, k2_ref, k3_ref)
        for i in range(NLAYERS):
            outs[i][...] = jnp.dot(u, wk_ref[i],
                                   preferred_element_type=jnp.float32)

    out_sd = jax.ShapeDtypeStruct((E, HID), jnp.float32)
    return pl.pallas_call(
        body,
        grid=grid,
        in_specs=[
            pl.BlockSpec((EB, HID), lambda i: (i, 0)),
            pl.BlockSpec((8, BASIS), lambda i: (0, 0)),
            pl.BlockSpec((1, BASIS), lambda i: (0, 0)),
            pl.BlockSpec((BASIS, BASIS), lambda i: (0, 0)),
            pl.BlockSpec((1, BASIS), lambda i: (0, 0)),
            pl.BlockSpec((NLAYERS, BASIS, HID), lambda i: (0, 0, 0)),
        ],
        out_specs=[pl.BlockSpec((EB, HID), lambda i: (i, 0))] * NLAYERS,
        out_shape=[out_sd] * NLAYERS,
        compiler_params=pltpu.CompilerParams(
            dimension_semantics=("parallel",)),
    )(diff, Wb1, bb1, Wb2, bb2, Wk)


def _tc_embed(x, W_embed):
    def body(x_ref, w_ref, o_ref):
        o_ref[...] = jnp.dot(x_ref[...], w_ref[...],
                             preferred_element_type=jnp.float32)

    return pl.pallas_call(
        body, out_shape=jax.ShapeDtypeStruct((N, HID), jnp.float32)
    )(x, W_embed)


def _tc_node_update(aggs, h, gamma, beta, W1, b1, W2, b2, Wro=None, bro=None):
    readout = Wro is not None
    NB = 1000
    grid = (N // NB,)

    def body(agga_ref, aggb_ref, h_ref, g_ref, be_ref, w1_ref, b1_ref,
             w2_ref, b2_ref, *rest):
        out_ref = rest[-1]
        agg = agga_ref[...] + aggb_ref[...]
        mu = jnp.mean(agg, axis=-1, keepdims=True)
        xc = agg - mu
        var = jnp.mean(xc * xc, axis=-1, keepdims=True)
        z = xc / jnp.sqrt(var + 1e-5) * g_ref[...] + be_ref[...]
        z = _gelu(jnp.dot(z, w1_ref[...], preferred_element_type=jnp.float32)
                  + b1_ref[...])
        z = jnp.dot(z, w2_ref[...], preferred_element_type=jnp.float32) + b2_ref[...]
        hn = h_ref[...] + z
        if readout:
            wro_ref, bro_ref = rest[0], rest[1]
            out_ref[...] = (jnp.dot(hn, wro_ref[...],
                                    preferred_element_type=jnp.float32)
                            + bro_ref[...])
        else:
            out_ref[...] = hn

    full = lambda i: (0, 0)
    in_specs = [
        pl.BlockSpec((NB, HID), lambda i: (i, 0)),
        pl.BlockSpec((NB, HID), lambda i: (i + N // NB, 0)),
        pl.BlockSpec((NB, HID), lambda i: (i, 0)),
        pl.BlockSpec((1, HID), full),
        pl.BlockSpec((1, HID), full),
        pl.BlockSpec((HID, WIDE * HID), full),
        pl.BlockSpec((1, WIDE * HID), full),
        pl.BlockSpec((WIDE * HID, HID), full),
        pl.BlockSpec((1, HID), full),
    ]
    args = [aggs, aggs, h, gamma, beta, W1, b1, W2, b2]
    odim = D if readout else HID
    if readout:
        in_specs += [pl.BlockSpec((HID, D), full), pl.BlockSpec((1, D), full)]
        args += [Wro, bro]
    return pl.pallas_call(
        body,
        grid=grid,
        in_specs=in_specs,
        out_specs=pl.BlockSpec((NB, odim), lambda i: (i, 0)),
        out_shape=jax.ShapeDtypeStruct((N, odim), jnp.float32),
        compiler_params=pltpu.CompilerParams(
            dimension_semantics=("parallel",)),
    )(*args)


def kernel(x, pos, edge_index, batch, W_embed, Wb1, bb1, Wb2, bb2, Wk,
           gamma, beta, W1, b1, W2, b2, Wro, bro):
    send = edge_index[0].astype(jnp.int32)
    recv = edge_index[1].astype(jnp.int32)
    pos128 = jnp.zeros((N, HID), jnp.float32).at[:, :3].set(pos)
    Wb1p = jnp.zeros((8, BASIS), jnp.float32).at[:3].set(Wb1)

    diff = _sc_pos_diff(pos128, send, recv)
    kerns = _tc_edge_basis(diff, Wb1p, bb1[None, :], Wb2, bb2[None, :], Wk)
    h = _tc_embed(x, W_embed)
    for i in range(NLAYERS):
        aggs = _sc_conv(h, kerns[i], send, recv)
        last = i == NLAYERS - 1
        h = _tc_node_update(
            aggs, h, gamma[i][None, :], beta[i][None, :],
            W1[i], b1[i][None, :], W2[i], b2[i][None, :],
            Wro if last else None, bro[None, :] if last else None)
    return h
